# Initial kernel scaffold; baseline (speedup 1.0000x reference)
import functools

import jax
import jax.numpy as jnp
from jax import lax
from jax.experimental import pallas as pl
from jax.experimental.pallas import tpu as pltpu
from jax.experimental.pallas import tpu_sc as plsc

N = 10000
E = 320000
D = 128

NC = 2
NS = 16
NW = NC * NS
L = 16

EPT = E // NW
K = 80
NCH = EPT // K
RPT = N // NS
ZR = 125

_NEG = -3.0e38


def _tc_prep_body(x_ref, w_ref, al_ref, ar_ref, z_ref, el_ref, er_ref):
    z = jnp.dot(x_ref[...], w_ref[...], preferred_element_type=jnp.float32)
    z_ref[...] = z
    el_ref[...] = jnp.sum(z * al_ref[...], axis=1, keepdims=True)
    er_ref[...] = jnp.sum(z * ar_ref[...], axis=1, keepdims=True)


def _tc_prep(x, w, al, ar):
    return pl.pallas_call(
        _tc_prep_body,
        out_shape=(
            jax.ShapeDtypeStruct((N, D), jnp.float32),
            jax.ShapeDtypeStruct((N, 1), jnp.float32),
            jax.ShapeDtypeStruct((N, 1), jnp.float32),
        ),
    )(x, w, al, ar)


def _tc_mid_body(acc_ref, asum_ref, b_ref, g_ref, be_ref, w_ref, al_ref,
                 ar_ref, z_ref, el_ref, er_ref):
    s = jnp.sum(asum_ref[...], axis=0)
    h = (acc_ref[0] + acc_ref[1]) / (s + 1e-9) + b_ref[...]
    mu = jnp.mean(h, axis=1, keepdims=True)
    var = jnp.mean((h - mu) ** 2, axis=1, keepdims=True)
    h = (h - mu) / jnp.sqrt(var + 1e-5) * g_ref[...] + be_ref[...]
    h = jnp.maximum(h, 0.0)
    z = jnp.dot(h, w_ref[...], preferred_element_type=jnp.float32)
    z_ref[...] = z
    el_ref[...] = jnp.sum(z * al_ref[...], axis=1, keepdims=True)
    er_ref[...] = jnp.sum(z * ar_ref[...], axis=1, keepdims=True)


def _tc_mid(acc, asum3, b, gamma, beta, w, al, ar):
    return pl.pallas_call(
        _tc_mid_body,
        out_shape=(
            jax.ShapeDtypeStruct((N, D), jnp.float32),
            jax.ShapeDtypeStruct((N, 1), jnp.float32),
            jax.ShapeDtypeStruct((N, 1), jnp.float32),
        ),
    )(acc, asum3, b, gamma, beta, w, al, ar)


def _tc_final_body(acc_ref, asum_ref, b_ref, out_ref):
    s = jnp.sum(asum_ref[...], axis=0)
    out_ref[...] = (acc_ref[0] + acc_ref[1]) / (s + 1e-9) + b_ref[...]


def _tc_final(acc, asum3, b):
    return pl.pallas_call(
        _tc_final_body,
        out_shape=jax.ShapeDtypeStruct((N, D), jnp.float32),
    )(acc, asum3, b)


def _sc_edge_body(src_hbm, dst3_hbm, el_hbm, er_hbm, z_hbm,
                  acc_out, asum_out,
                  src_v, dst3_v, el_v, er_v, asum_v, w_v, rowbuf, zbuf,
                  acc_sh):
    cid = lax.axis_index("c")
    sid = lax.axis_index("s")
    wid = cid * NS + sid
    base = wid * EPT

    zero16 = jnp.zeros((L,), jnp.float32)

    pltpu.sync_copy(src_hbm.at[pl.ds(base, EPT)], src_v)
    pltpu.sync_copy(dst3_hbm.at[wid], dst3_v)
    pltpu.sync_copy(el_hbm, el_v)
    pltpu.sync_copy(er_hbm, er_v)

    def _zb(r, carry):
        for k in range(D // L):
            zbuf[r, pl.ds(k * L, L)] = zero16
        return carry
    lax.fori_loop(0, ZR, _zb, 0)

    def _za(i, carry):
        asum_v[pl.ds(i * L, L)] = zero16
        return carry
    lax.fori_loop(0, N // L, _za, 0)

    for k5 in range(RPT // ZR):
        pltpu.sync_copy(zbuf, acc_sh.at[pl.ds(sid * RPT + k5 * ZR, ZR)])
    plsc.subcore_barrier()

    def _mx(i, m):
        return jnp.maximum(m, el_v[pl.ds(i * L, L)])
    mvec = lax.fori_loop(0, N // L, _mx, jnp.full((L,), _NEG, jnp.float32))
    elmax = jnp.max(mvec)
    elmax_v = jnp.full((L,), elmax, jnp.float32)

    def _chunk(ch, carry):
        off = ch * K
        pltpu.sync_copy(z_hbm.at[src_v.at[pl.ds(off, K)]], rowbuf)
        for v in range(K // L):
            s = src_v[pl.ds(off + v * L, L)]
            d = dst3_v[ch, pl.ds(v * L, L)]
            els = plsc.load_gather(el_v, [s])
            erd = plsc.load_gather(er_v, [d])
            u = els + erd
            ev = jnp.maximum(u, 0.2 * u)
            t = erd + elmax_v
            cv = jnp.maximum(t, 0.2 * t)
            w = jnp.exp(ev - cv)
            plsc.addupdate_scatter(asum_v, [d], w)
            w_v[pl.ds(v * L, L)] = w
        def _srow(j, c2):
            wj = w_v[j]
            for k in range(D // L):
                sl = pl.ds(k * L, L)
                rowbuf[j, sl] = rowbuf[j, sl] * wj
            return c2
        lax.fori_loop(0, K, _srow, 0)
        pltpu.sync_copy(rowbuf, acc_sh.at[dst3_v.at[ch]], add=True)
        return carry
    lax.fori_loop(0, NCH, _chunk, 0)

    plsc.subcore_barrier()

    pltpu.sync_copy(asum_v, asum_out.at[wid])
    for k5 in range(RPT // ZR):
        r0 = sid * RPT + k5 * ZR
        pltpu.sync_copy(acc_sh.at[pl.ds(r0, ZR)], zbuf)
        pltpu.sync_copy(zbuf, acc_out.at[cid, pl.ds(r0, ZR)])


_sc_edge = functools.partial(
    pl.kernel,
    out_type=(
        jax.ShapeDtypeStruct((NC, N, D), jnp.float32),
        jax.ShapeDtypeStruct((NW, N), jnp.float32),
    ),
    mesh=plsc.VectorSubcoreMesh(core_axis_name="c", subcore_axis_name="s",
                                num_cores=NC, num_subcores=NS),
    scratch_types=[
        pltpu.VMEM((EPT,), jnp.int32),
        pltpu.VMEM((NCH, K), jnp.int32),
        pltpu.VMEM((N,), jnp.float32),
        pltpu.VMEM((N,), jnp.float32),
        pltpu.VMEM((N,), jnp.float32),
        pltpu.VMEM((K,), jnp.float32),
        pltpu.VMEM((K, D), jnp.float32),
        pltpu.VMEM((ZR, D), jnp.float32),
        pltpu.VMEM_SHARED((N, D), jnp.float32),
    ],
)(_sc_edge_body)


def kernel(feat, edge_index, W0, al0, ar0, b0, gamma, beta, W1, al1, ar1, b1):
    src = edge_index[0]
    dst = edge_index[1]
    dst3 = dst.reshape(NW, NCH, K)

    z0, el0, er0 = _tc_prep(feat, W0, al0, ar0)
    acc0, asum0 = _sc_edge(src, dst3, el0.reshape(N), er0.reshape(N), z0)
    z1, el1, er1 = _tc_mid(acc0, asum0.reshape(NW, N, 1), b0.reshape(1, D),
                           gamma.reshape(1, D), beta.reshape(1, D),
                           W1, al1, ar1)
    acc1, asum1 = _sc_edge(src, dst3, el1.reshape(N), er1.reshape(N), z1)
    return _tc_final(acc1, asum1.reshape(NW, N, 1), b1.reshape(1, D))


# speedup vs baseline: 14.9823x; 14.9823x over previous
import functools

import jax
import jax.numpy as jnp
from jax import lax
from jax.experimental import pallas as pl
from jax.experimental.pallas import tpu as pltpu
from jax.experimental.pallas import tpu_sc as plsc

N = 10000
E = 320000
D = 128

NC = 2
NS = 16
NW = NC * NS
L = 16

EPT = E // NW
K = 128
EPTP = 10240
BLK = 8
BLKE = BLK * K
NBLK = EPTP // BLKE

_NEG = -3.0e38


def _tc_prep_body(x_ref, w_ref, al_ref, ar_ref, z_ref, el_ref, er_ref):
    z = jnp.dot(x_ref[...], w_ref[...], preferred_element_type=jnp.float32)
    z_ref[...] = z
    el_ref[...] = jnp.sum(z * al_ref[...], axis=1, keepdims=True)
    er_ref[...] = jnp.sum(z * ar_ref[...], axis=1, keepdims=True)


def _tc_prep(x, w, al, ar):
    return pl.pallas_call(
        _tc_prep_body,
        out_shape=(
            jax.ShapeDtypeStruct((N, D), jnp.float32),
            jax.ShapeDtypeStruct((N, 1), jnp.float32),
            jax.ShapeDtypeStruct((N, 1), jnp.float32),
        ),
    )(x, w, al, ar)


def _tc_mid_body(acc_ref, asum_ref, b_ref, g_ref, be_ref, w_ref, al_ref,
                 ar_ref, z_ref, el_ref, er_ref):
    s = jnp.sum(asum_ref[...], axis=0)
    h = (acc_ref[0] + acc_ref[1]) / (s + 1e-9) + b_ref[...]
    mu = jnp.mean(h, axis=1, keepdims=True)
    var = jnp.mean((h - mu) ** 2, axis=1, keepdims=True)
    h = (h - mu) / jnp.sqrt(var + 1e-5) * g_ref[...] + be_ref[...]
    h = jnp.maximum(h, 0.0)
    z = jnp.dot(h, w_ref[...], preferred_element_type=jnp.float32)
    z_ref[...] = z
    el_ref[...] = jnp.sum(z * al_ref[...], axis=1, keepdims=True)
    er_ref[...] = jnp.sum(z * ar_ref[...], axis=1, keepdims=True)


def _tc_mid(acc, asum3, b, gamma, beta, w, al, ar):
    return pl.pallas_call(
        _tc_mid_body,
        out_shape=(
            jax.ShapeDtypeStruct((N, D), jnp.float32),
            jax.ShapeDtypeStruct((N, 1), jnp.float32),
            jax.ShapeDtypeStruct((N, 1), jnp.float32),
        ),
    )(acc, asum3, b, gamma, beta, w, al, ar)


def _tc_final_body(acc_ref, asum_ref, b_ref, out_ref):
    s = jnp.sum(asum_ref[...], axis=0)
    out_ref[...] = (acc_ref[0] + acc_ref[1]) / (s + 1e-9) + b_ref[...]


def _tc_final(acc, asum3, b):
    return pl.pallas_call(
        _tc_final_body,
        out_shape=jax.ShapeDtypeStruct((N, D), jnp.float32),
    )(acc, asum3, b)


def _sc_edge_body(src_hbm, dst4_hbm, el_hbm, er_hbm, z_hbm, zeros_hbm,
                  zerosn_hbm, acc_out, asum_out,
                  srcb, dstb, el_v, er_v, w_v, rowbuf,
                  acc_sh, asum_sh):
    cid = lax.axis_index("c")
    sid = lax.axis_index("s")
    wid = cid * NS + sid

    zero16 = jnp.zeros((L,), jnp.float32)

    pltpu.sync_copy(el_hbm, el_v)
    pltpu.sync_copy(er_hbm, er_v)

    @pl.when(sid == 0)
    def _zero_acc():
        pltpu.sync_copy(zeros_hbm, acc_sh)
    @pl.when(sid == 1)
    def _zero_asum():
        pltpu.sync_copy(zerosn_hbm, asum_sh)
    plsc.subcore_barrier()

    def _mx(i, m):
        return jnp.maximum(m, el_v[pl.ds(i * L, L)])
    mvec = lax.fori_loop(0, N // L, _mx, jnp.full((L,), _NEG, jnp.float32))
    lane = lax.iota(jnp.int32, L)
    for shift in (1, 2, 4, 8):
        w_v[pl.ds(0, L)] = mvec
        mvec = jnp.maximum(mvec, plsc.load_gather(w_v, [lane ^ shift]))
    elmax_v = mvec

    def _block(blk, carry):
        pltpu.sync_copy(
            src_hbm.at[pl.ds(wid * EPTP + blk * BLKE, BLKE)], srcb)
        pltpu.sync_copy(dst4_hbm.at[wid, blk], dstb)

        def _chunk(cc, c1):
            pltpu.sync_copy(z_hbm.at[srcb.at[pl.ds(cc * K, K)]], rowbuf)
            for v in range(K // L):
                s = srcb[pl.ds(cc * K + v * L, L)]
                d = dstb[cc, pl.ds(v * L, L)]
                els = plsc.load_gather(el_v, [s])
                erd = plsc.load_gather(er_v, [d])
                u = els + erd
                ev = jnp.maximum(u, 0.2 * u)
                t = erd + elmax_v
                cv = jnp.maximum(t, 0.2 * t)
                w = jnp.exp(ev - cv)
                pos = lane + (blk * BLKE + cc * K + v * L)
                w = jnp.where(pos < EPT, w, 0.0)
                w_v[pl.ds(v * L, L)] = w
            def _srow(j, c2):
                wj = w_v[pl.ds(j, L)][0]
                for k in range(D // L):
                    sl = pl.ds(k * L, L)
                    rowbuf[j, sl] = rowbuf[j, sl] * wj
                return c2
            lax.fori_loop(0, K, _srow, 0)
            pltpu.sync_copy(w_v.at[pl.ds(0, K)], asum_sh.at[dstb.at[cc]],
                            add=True)
            pltpu.sync_copy(rowbuf, acc_sh.at[dstb.at[cc]], add=True)
            return c1
        lax.fori_loop(0, BLK, _chunk, 0)
        return carry
    lax.fori_loop(0, NBLK, _block, 0)

    plsc.subcore_barrier()

    @pl.when(sid == 0)
    def _wb():
        pltpu.sync_copy(asum_sh, el_v)
        pltpu.sync_copy(el_v, asum_out.at[pl.ds(cid * N, N)])
        pltpu.sync_copy(acc_sh, acc_out.at[cid])


_sc_edge = functools.partial(
    pl.kernel,
    out_type=(
        jax.ShapeDtypeStruct((NC, N, D), jnp.float32),
        jax.ShapeDtypeStruct((NC * N,), jnp.float32),
    ),
    mesh=plsc.VectorSubcoreMesh(core_axis_name="c", subcore_axis_name="s",
                                num_cores=NC, num_subcores=NS),
    compiler_params=pltpu.CompilerParams(needs_layout_passes=False),
    scratch_types=[
        pltpu.VMEM((BLKE,), jnp.int32),
        pltpu.VMEM((BLK, K), jnp.int32),
        pltpu.VMEM((N,), jnp.float32),
        pltpu.VMEM((N,), jnp.float32),
        pltpu.VMEM((K + L,), jnp.float32),
        pltpu.VMEM((K, D), jnp.float32),
        pltpu.VMEM_SHARED((N, D), jnp.float32),
        pltpu.VMEM_SHARED((N,), jnp.float32),
    ],
)(_sc_edge_body)


def kernel(feat, edge_index, W0, al0, ar0, b0, gamma, beta, W1, al1, ar1, b1):
    ei = edge_index.reshape(2, NW, EPT)
    eip = jnp.pad(ei, ((0, 0), (0, 0), (0, EPTP - EPT)))
    src = eip[0].reshape(NW * EPTP)
    dst4 = eip[1].reshape(NW, NBLK, BLK, K)
    zeros = jnp.zeros((N, D), jnp.float32)
    zerosn = jnp.zeros((N,), jnp.float32)

    z0, el0, er0 = _tc_prep(feat, W0, al0, ar0)
    acc0, asum0 = _sc_edge(src, dst4, el0.reshape(N), er0.reshape(N), z0,
                           zeros, zerosn)
    z1, el1, er1 = _tc_mid(acc0, asum0.reshape(NC, N, 1), b0.reshape(1, D),
                           gamma.reshape(1, D), beta.reshape(1, D),
                           W1, al1, ar1)
    acc1, asum1 = _sc_edge(src, dst4, el1.reshape(N), er1.reshape(N), z1,
                           zeros, zerosn)
    return _tc_final(acc1, asum1.reshape(NC, N, 1), b1.reshape(1, D))


# speedup vs baseline: 19.5223x; 1.3030x over previous
import functools

import jax
import jax.numpy as jnp
from jax import lax
from jax.experimental import pallas as pl
from jax.experimental.pallas import tpu as pltpu
from jax.experimental.pallas import tpu_sc as plsc

N = 10000
E = 320000
D = 128

NC = 2
NS = 16
NW = NC * NS
L = 16

EPT = E // NW
K = 128
EPTP = 10240
BLK = 8
BLKE = BLK * K
NCH = EPTP // K


def _tc_prep_body(x_ref, w_ref, al_ref, ar_ref, z_ref, el_ref, er_ref,
                  em_ref):
    z = jnp.dot(x_ref[...], w_ref[...], preferred_element_type=jnp.float32)
    z_ref[...] = z
    el = jnp.sum(z * al_ref[...], axis=1, keepdims=True)
    el_ref[...] = el
    er_ref[...] = jnp.sum(z * ar_ref[...], axis=1, keepdims=True)
    em_ref[...] = jnp.full((1, L), jnp.max(el), jnp.float32)


def _tc_prep(x, w, al, ar):
    return pl.pallas_call(
        _tc_prep_body,
        out_shape=(
            jax.ShapeDtypeStruct((N, D), jnp.float32),
            jax.ShapeDtypeStruct((N, 1), jnp.float32),
            jax.ShapeDtypeStruct((N, 1), jnp.float32),
            jax.ShapeDtypeStruct((1, L), jnp.float32),
        ),
    )(x, w, al, ar)


def _tc_mid_body(acc_ref, asum_ref, b_ref, g_ref, be_ref, w_ref, al_ref,
                 ar_ref, z_ref, el_ref, er_ref, em_ref):
    s = jnp.sum(asum_ref[...], axis=0)
    h = (acc_ref[0] + acc_ref[1]) / (s + 1e-9) + b_ref[...]
    mu = jnp.mean(h, axis=1, keepdims=True)
    var = jnp.mean((h - mu) ** 2, axis=1, keepdims=True)
    h = (h - mu) / jnp.sqrt(var + 1e-5) * g_ref[...] + be_ref[...]
    h = jnp.maximum(h, 0.0)
    z = jnp.dot(h, w_ref[...], preferred_element_type=jnp.float32)
    z_ref[...] = z
    el = jnp.sum(z * al_ref[...], axis=1, keepdims=True)
    el_ref[...] = el
    er_ref[...] = jnp.sum(z * ar_ref[...], axis=1, keepdims=True)
    em_ref[...] = jnp.full((1, L), jnp.max(el), jnp.float32)


def _tc_mid(acc, asum3, b, gamma, beta, w, al, ar):
    return pl.pallas_call(
        _tc_mid_body,
        out_shape=(
            jax.ShapeDtypeStruct((N, D), jnp.float32),
            jax.ShapeDtypeStruct((N, 1), jnp.float32),
            jax.ShapeDtypeStruct((N, 1), jnp.float32),
            jax.ShapeDtypeStruct((1, L), jnp.float32),
        ),
    )(acc, asum3, b, gamma, beta, w, al, ar)


def _tc_final_body(acc_ref, asum_ref, b_ref, out_ref):
    s = jnp.sum(asum_ref[...], axis=0)
    out_ref[...] = (acc_ref[0] + acc_ref[1]) / (s + 1e-9) + b_ref[...]


def _tc_final(acc, asum3, b):
    return pl.pallas_call(
        _tc_final_body,
        out_shape=jax.ShapeDtypeStruct((N, D), jnp.float32),
    )(acc, asum3, b)


def _sc_edge_body(src_hbm, dst4_hbm, el_hbm, er_hbm, emax_hbm, z_hbm,
                  zeros_hbm, zerosn_hbm,
                  acc_out, asum_out,
                  srcb, dstb, elb, erb, wbuf, rowbuf, emax_s, asumstg,
                  acc_sh, asum_sh, gsem0, gsem1, ssem0, ssem1):
    cid = lax.axis_index("c")
    sid = lax.axis_index("s")
    wid = cid * NS + sid
    ebase = wid * EPTP

    pltpu.sync_copy(emax_hbm, emax_s)

    @pl.when(sid == 0)
    def _zero_acc():
        pltpu.sync_copy(zeros_hbm, acc_sh)

    @pl.when(sid == 1)
    def _zero_asum():
        pltpu.sync_copy(zerosn_hbm, asum_sh)
    plsc.subcore_barrier()

    emaxv = emax_s[pl.ds(0, L)]
    lane = lax.iota(jnp.int32, L)
    gsems = (gsem0, gsem1)
    ssems = (ssem0, ssem1)

    def stage_block(b):
        bp = b % 2
        pltpu.sync_copy(src_hbm.at[pl.ds(ebase + b * BLKE, BLKE)],
                        srcb.at[bp])
        pltpu.sync_copy(dst4_hbm.at[wid, b], dstb.at[bp])

    def src_idx(cc):
        bp = (cc // BLK) % 2
        return srcb.at[bp, pl.ds((cc % BLK) * K, K)]

    def dst_idx(cc):
        bp = (cc // BLK) % 2
        return dstb.at[bp, cc % BLK]

    def issue_gathers(cc, p):
        pltpu.async_copy(z_hbm.at[src_idx(cc)], rowbuf.at[p], gsems[p])
        pltpu.async_copy(el_hbm.at[src_idx(cc)], elb.at[p], gsems[p])
        pltpu.async_copy(er_hbm.at[dst_idx(cc)], erb.at[p], gsems[p])

    def wait_gathers(cc, p):
        pltpu.make_async_copy(z_hbm.at[src_idx(cc)], rowbuf.at[p],
                              gsems[p]).wait()
        pltpu.make_async_copy(el_hbm.at[src_idx(cc)], elb.at[p],
                              gsems[p]).wait()
        pltpu.make_async_copy(er_hbm.at[dst_idx(cc)], erb.at[p],
                              gsems[p]).wait()

    def issue_scatters(cc, p):
        pltpu.async_copy(wbuf.at[p], asum_sh.at[dst_idx(cc)], ssems[p],
                         add=True)
        pltpu.async_copy(rowbuf.at[p], acc_sh.at[dst_idx(cc)], ssems[p],
                         add=True)

    def wait_scatters(cc, p):
        pltpu.make_async_copy(wbuf.at[p], asum_sh.at[dst_idx(cc)],
                              ssems[p]).wait()
        pltpu.make_async_copy(rowbuf.at[p], acc_sh.at[dst_idx(cc)],
                              ssems[p]).wait()

    def compute_scale(cc, p):
        offv = cc * K
        for v in range(K // L):
            els = elb[p, pl.ds(v * L, L)]
            erd = erb[p, pl.ds(v * L, L)]
            u = els + erd
            ev = jnp.maximum(u, 0.2 * u)
            t = erd + emaxv
            cv = jnp.maximum(t, 0.2 * t)
            w = jnp.exp(ev - cv)
            pos = lane + (offv + v * L)
            w = jnp.where(pos < EPT, w, 0.0)
            wbuf[p, pl.ds(v * L, L)] = w

        def _svb(vb, c2):
            wv = wbuf[p, pl.ds(vb * L, L)]
            for l in range(L):
                j = vb * L + l
                wj = wv[l]
                for k in range(D // L):
                    sl = pl.ds(k * L, L)
                    rowbuf[p, j, sl] = rowbuf[p, j, sl] * wj
            return c2
        lax.fori_loop(0, K // L, _svb, 0)

    stage_block(0)
    issue_gathers(0, 0)

    def _pair(i, carry):
        for half in (0, 1):
            cc = 2 * i + half
            p = half
            q = 1 - half

            @pl.when(cc >= 1)
            def _ws(cc=cc, q=q):
                wait_scatters(cc - 1, q)

            @pl.when(jnp.logical_and(cc + 1 < NCH, (cc + 1) % BLK == 0))
            def _sb(cc=cc):
                stage_block((cc + 1) // BLK)

            @pl.when(cc + 1 < NCH)
            def _ig(cc=cc, q=q):
                issue_gathers(cc + 1, q)

            wait_gathers(cc, p)
            compute_scale(cc, p)
            issue_scatters(cc, p)
        return carry
    lax.fori_loop(0, NCH // 2, _pair, 0)

    wait_scatters(NCH - 1, 1)
    plsc.subcore_barrier()

    @pl.when(sid == 0)
    def _wb():
        pltpu.sync_copy(asum_sh, asumstg)
        pltpu.sync_copy(asumstg, asum_out.at[pl.ds(cid * N, N)])
        pltpu.sync_copy(acc_sh, acc_out.at[cid])


_sc_edge = functools.partial(
    pl.kernel,
    out_type=(
        jax.ShapeDtypeStruct((NC, N, D), jnp.float32),
        jax.ShapeDtypeStruct((NC * N,), jnp.float32),
    ),
    mesh=plsc.VectorSubcoreMesh(core_axis_name="c", subcore_axis_name="s",
                                num_cores=NC, num_subcores=NS),
    compiler_params=pltpu.CompilerParams(needs_layout_passes=False),
    scratch_types=[
        pltpu.VMEM((2, BLKE), jnp.int32),
        pltpu.VMEM((2, BLK, K), jnp.int32),
        pltpu.VMEM((2, K), jnp.float32),
        pltpu.VMEM((2, K), jnp.float32),
        pltpu.VMEM((2, K), jnp.float32),
        pltpu.VMEM((2, K, D), jnp.float32),
        pltpu.VMEM((L,), jnp.float32),
        pltpu.VMEM((N,), jnp.float32),
        pltpu.VMEM_SHARED((N, D), jnp.float32),
        pltpu.VMEM_SHARED((N,), jnp.float32),
        pltpu.SemaphoreType.DMA,
        pltpu.SemaphoreType.DMA,
        pltpu.SemaphoreType.DMA,
        pltpu.SemaphoreType.DMA,
    ],
)(_sc_edge_body)


def kernel(feat, edge_index, W0, al0, ar0, b0, gamma, beta, W1, al1, ar1, b1):
    ei = edge_index.reshape(2, NW, EPT)
    eip = jnp.pad(ei, ((0, 0), (0, 0), (0, EPTP - EPT)))
    src = eip[0].reshape(NW * EPTP)
    dst4 = eip[1].reshape(NW, NCH // BLK, BLK, K)
    zeros = jnp.zeros((N, D), jnp.float32)
    zerosn = jnp.zeros((N,), jnp.float32)

    z0, el0, er0, em0 = _tc_prep(feat, W0, al0, ar0)
    acc0, asum0 = _sc_edge(src, dst4, el0.reshape(N), er0.reshape(N),
                           em0.reshape(L), z0, zeros, zerosn)
    z1, el1, er1, em1 = _tc_mid(acc0, asum0.reshape(NC, N, 1),
                                b0.reshape(1, D), gamma.reshape(1, D),
                                beta.reshape(1, D), W1, al1, ar1)
    acc1, asum1 = _sc_edge(src, dst4, el1.reshape(N), er1.reshape(N),
                           em1.reshape(L), z1, zeros, zerosn)
    return _tc_final(acc1, asum1.reshape(NC, N, 1), b1.reshape(1, D))
